# Initial kernel scaffold; baseline (speedup 1.0000x reference)
import jax
import jax.numpy as jnp
from jax.experimental import pallas as pl


def kernel(x, edge_index, W1, b1, W2, b2, W3, b3):
    raise NotImplementedError("write your pallas kernel here")


# speedup vs baseline: 16.1656x; 16.1656x over previous
import functools

import jax
import jax.numpy as jnp
from jax import lax
from jax.experimental import pallas as pl
from jax.experimental.pallas import tpu as pltpu
from jax.experimental.pallas import tpu_sc as plsc

NCORES = 2
NSUB = 16
NTILES = NCORES * NSUB
LANES = 128
KJ = 4
BLK = 1000


def _pad_to(v, m):
    return -(-v // m) * m


@functools.lru_cache(None)
def _make_degree(np_rows, ebrows):
    rows_per_tile = ebrows // NTILES
    nblk = rows_per_tile // KJ
    stripe = np_rows // NSUB
    mesh = plsc.VectorSubcoreMesh(core_axis_name="c", subcore_axis_name="s")

    def body(dst_hbm, z_hbm, ones_hbm, out_hbm, idxd, ones_v, acc):
        c = lax.axis_index("c")
        s = lax.axis_index("s")
        row0 = s * stripe
        tbase = (c * NSUB + s) * rows_per_tile
        pltpu.sync_copy(ones_hbm, ones_v)
        pltpu.sync_copy(z_hbm, acc.at[pl.ds(row0, stripe)])
        plsc.subcore_barrier()

        def blk(i, carry):
            r = tbase + i * KJ
            pltpu.sync_copy(dst_hbm.at[pl.ds(r, KJ)], idxd)
            for j in range(KJ):
                pltpu.sync_copy(ones_v, acc.at[idxd.at[j]], add=True)
            return carry

        lax.fori_loop(0, nblk, blk, 0)
        plsc.subcore_barrier()
        pltpu.sync_copy(acc.at[pl.ds(row0, stripe)],
                        out_hbm.at[c, pl.ds(row0, stripe)])

    return pl.kernel(
        body,
        mesh=mesh,
        compiler_params=pltpu.CompilerParams(use_tc_tiling_on_sc=False),
        out_type=jax.ShapeDtypeStruct((NCORES, np_rows, 1), jnp.float32),
        scratch_types=[
            pltpu.VMEM((KJ, LANES), jnp.int32),
            pltpu.VMEM((LANES, 1), jnp.float32),
            pltpu.VMEM_SHARED((np_rows, 1), jnp.float32),
        ],
    )


@functools.lru_cache(None)
def _make_scatter(np_rows, nch, c_width, ebrows):
    rows_per_tile = ebrows // NTILES
    nblk = rows_per_tile // KJ
    stripe = np_rows // NSUB
    mesh = plsc.VectorSubcoreMesh(core_axis_name="c", subcore_axis_name="s")

    def body(*refs):
        g_refs = refs[:nch]
        src_hbm, dst_hbm, z_hbm, out_hbm = refs[nch:nch + 4]
        idxs, idxd, rows, acc, sem = refs[nch + 4:]
        c = lax.axis_index("c")
        s = lax.axis_index("s")
        row0 = s * stripe
        tbase = (c * NSUB + s) * rows_per_tile
        for ck in range(nch):
            pltpu.sync_copy(z_hbm, acc.at[pl.ds(row0, stripe)])
            plsc.subcore_barrier()

            def blk(i, carry, ck=ck):
                r = tbase + i * KJ
                pltpu.sync_copy(src_hbm.at[pl.ds(r, KJ)], idxs)
                pltpu.sync_copy(dst_hbm.at[pl.ds(r, KJ)], idxd)
                descs = [
                    pltpu.async_copy(g_refs[ck].at[idxs.at[j]],
                                     rows.at[pl.ds(j * LANES, LANES)], sem)
                    for j in range(KJ)
                ]
                for d in descs:
                    d.wait()
                for j in range(KJ):
                    pltpu.sync_copy(rows.at[pl.ds(j * LANES, LANES)],
                                    acc.at[idxd.at[j]], add=True)
                return carry

            lax.fori_loop(0, nblk, blk, 0)
            plsc.subcore_barrier()
            pltpu.sync_copy(acc.at[pl.ds(row0, stripe)],
                            out_hbm.at[c, ck, pl.ds(row0, stripe)])

    return pl.kernel(
        body,
        mesh=mesh,
        compiler_params=pltpu.CompilerParams(use_tc_tiling_on_sc=False),
        out_type=jax.ShapeDtypeStruct((NCORES, nch, np_rows, c_width),
                                      jnp.float32),
        scratch_types=[
            pltpu.VMEM((KJ, LANES), jnp.int32),
            pltpu.VMEM((KJ, LANES), jnp.int32),
            pltpu.VMEM((KJ * LANES, c_width), jnp.float32),
            pltpu.VMEM_SHARED((np_rows, c_width), jnp.float32),
            pltpu.SemaphoreType.DMA,
        ],
    )


def _tc_prep(deg2, x):
    n, f = x.shape
    grid = n // BLK

    def body(deg_ref, x_ref, dinv_ref, g1_ref):
        d = deg_ref[0, :, 0] + deg_ref[1, :, 0] + 1.0
        dv = lax.rsqrt(d)
        dinv_ref[:, 0] = dv
        g = x_ref[...] * dv[:, None]
        g1_ref[...] = jnp.concatenate(
            [g, jnp.zeros((BLK, 32 - f), jnp.float32)], axis=-1)

    return pl.pallas_call(
        body,
        grid=(grid,),
        in_specs=[pl.BlockSpec((NCORES, BLK, 1), lambda i: (0, i, 0)),
                  pl.BlockSpec((BLK, f), lambda i: (i, 0))],
        out_specs=[pl.BlockSpec((BLK, 1), lambda i: (i, 0)),
                   pl.BlockSpec((BLK, 32), lambda i: (i, 0))],
        out_shape=[jax.ShapeDtypeStruct((n, 1), jnp.float32),
                   jax.ShapeDtypeStruct((n, 32), jnp.float32)],
    )(deg2, x)


def _tc_layer1(s1, g1, dinv, w1p, b1r):
    n = g1.shape[0]
    grid = n // BLK

    def body(s_ref, g_ref, dv_ref, w_ref, b_ref, o0, o1, o2, o3):
        dv = dv_ref[:, 0]
        agg = (s_ref[0] + s_ref[1] + g_ref[...]) * dv[:, None]
        h = jnp.dot(agg, w_ref[...], preferred_element_type=jnp.float32,
                    precision=lax.Precision.HIGHEST) + b_ref[...]
        g2 = jnp.maximum(h, 0.0) * dv[:, None]
        o0[...] = g2[:, 0:32]
        o1[...] = g2[:, 32:64]
        o2[...] = g2[:, 64:96]
        o3[...] = g2[:, 96:128]

    return pl.pallas_call(
        body,
        grid=(grid,),
        in_specs=[pl.BlockSpec((NCORES, BLK, 32), lambda i: (0, i, 0)),
                  pl.BlockSpec((BLK, 32), lambda i: (i, 0)),
                  pl.BlockSpec((BLK, 1), lambda i: (i, 0)),
                  pl.BlockSpec((32, 128), lambda i: (0, 0)),
                  pl.BlockSpec((1, 128), lambda i: (0, 0))],
        out_specs=[pl.BlockSpec((BLK, 32), lambda i: (i, 0))] * 4,
        out_shape=[jax.ShapeDtypeStruct((n, 32), jnp.float32)] * 4,
    )(s1, g1, dinv, w1p, b1r)


def _tc_layer2(s2, g2s, dinv, w2, b2r, w3p):
    n = dinv.shape[0]
    grid = n // BLK

    def body(s_ref, g0, g1, g2, g3, dv_ref, w2_ref, b_ref, w3_ref, out_ref):
        dv = dv_ref[:, 0]
        gs = (g0, g1, g2, g3)
        parts = [s_ref[0, k] + s_ref[1, k] + gs[k][...] for k in range(4)]
        agg = jnp.concatenate(parts, axis=-1) * dv[:, None]
        h = jnp.dot(agg, w2_ref[...], preferred_element_type=jnp.float32,
                    precision=lax.Precision.HIGHEST) + b_ref[...]
        h = jnp.maximum(h, 0.0)
        p = jnp.dot(h, w3_ref[...], preferred_element_type=jnp.float32,
                    precision=lax.Precision.HIGHEST)
        out_ref[...] = p * dv[:, None]

    return pl.pallas_call(
        body,
        grid=(grid,),
        in_specs=[pl.BlockSpec((NCORES, 4, BLK, 32), lambda i: (0, 0, i, 0))]
        + [pl.BlockSpec((BLK, 32), lambda i: (i, 0))] * 4
        + [pl.BlockSpec((BLK, 1), lambda i: (i, 0)),
           pl.BlockSpec((128, 128), lambda i: (0, 0)),
           pl.BlockSpec((1, 128), lambda i: (0, 0)),
           pl.BlockSpec((128, 16), lambda i: (0, 0))],
        out_specs=pl.BlockSpec((BLK, 16), lambda i: (i, 0)),
        out_shape=jax.ShapeDtypeStruct((n, 16), jnp.float32),
    )(s2, *g2s, dinv, w2, b2r, w3p)


def _tc_out(s3, g3, dinv, b3r):
    n = dinv.shape[0]
    f_out = b3r.shape[1]
    grid = n // BLK

    def body(s_ref, g_ref, dv_ref, b_ref, out_ref):
        dv = dv_ref[:, 0]
        v = (s_ref[0] + s_ref[1] + g_ref[...]) * dv[:, None]
        out_ref[...] = v[:, :f_out] + b_ref[...]

    return pl.pallas_call(
        body,
        grid=(grid,),
        in_specs=[pl.BlockSpec((NCORES, BLK, 16), lambda i: (0, i, 0)),
                  pl.BlockSpec((BLK, 16), lambda i: (i, 0)),
                  pl.BlockSpec((BLK, 1), lambda i: (i, 0)),
                  pl.BlockSpec((1, f_out), lambda i: (0, 0))],
        out_specs=pl.BlockSpec((BLK, f_out), lambda i: (i, 0)),
        out_shape=jax.ShapeDtypeStruct((n, f_out), jnp.float32),
    )(s3, g3, dinv, b3r)


def kernel(x, edge_index, W1, b1, W2, b2, W3, b3):
    n, f_in = x.shape
    e = edge_index.shape[1]
    h = W1.shape[1]
    f_out = W3.shape[1]
    ep = _pad_to(e, NTILES * LANES)
    ebrows = ep // LANES
    pad = ep - e
    np_rows = _pad_to(n, NSUB * 8)
    stripe = np_rows // NSUB
    spread = jnp.arange(pad, dtype=jnp.int32) % NSUB
    src = jnp.concatenate([edge_index[0], spread]).reshape(ebrows, LANES)
    dst = jnp.concatenate([edge_index[1], n + spread]).reshape(ebrows, LANES)
    z1 = jnp.zeros((stripe, 1), jnp.float32)
    z32 = jnp.zeros((stripe, 32), jnp.float32)
    z16 = jnp.zeros((stripe, 16), jnp.float32)
    ones = jnp.ones((LANES, 1), jnp.float32)

    deg2 = _make_degree(np_rows, ebrows)(dst, z1, ones)
    dinv, g1 = _tc_prep(deg2, x)

    s1 = _make_scatter(np_rows, 1, 32, ebrows)(g1, src, dst, z32)
    w1p = jnp.concatenate([W1, jnp.zeros((32 - f_in, h), jnp.float32)], axis=0)
    g2s = _tc_layer1(s1[:, 0], g1, dinv, w1p, b1.reshape(1, h))

    s2 = _make_scatter(np_rows, 4, 32, ebrows)(*g2s, src, dst, z32)
    w3p = jnp.concatenate([W3, jnp.zeros((h, 16 - f_out), jnp.float32)], axis=1)
    g3 = _tc_layer2(s2, g2s, dinv, W2, b2.reshape(1, h), w3p)

    s3 = _make_scatter(np_rows, 1, 16, ebrows)(g3, src, dst, z16)
    return _tc_out(s3[:, 0], g3, dinv, b3.reshape(1, f_out))


# speedup vs baseline: 16.4417x; 1.0171x over previous
import functools

import jax
import jax.numpy as jnp
from jax import lax
from jax.experimental import pallas as pl
from jax.experimental.pallas import tpu as pltpu
from jax.experimental.pallas import tpu_sc as plsc

NCORES = 2
NSUB = 16
NTILES = NCORES * NSUB
LANES = 128
KJ = 4
BLK = 1000


def _pad_to(v, m):
    return -(-v // m) * m


@functools.lru_cache(None)
def _make_degree(np_rows, ebrows):
    rows_per_tile = ebrows // NTILES
    nblk = rows_per_tile // KJ
    stripe = np_rows // NSUB
    mesh = plsc.VectorSubcoreMesh(core_axis_name="c", subcore_axis_name="s")

    def body(dst_hbm, z_hbm, ones_hbm, out_hbm, idxd, ones_v, acc):
        c = lax.axis_index("c")
        s = lax.axis_index("s")
        row0 = s * stripe
        tbase = (c * NSUB + s) * rows_per_tile
        pltpu.sync_copy(ones_hbm, ones_v)
        pltpu.sync_copy(z_hbm, acc.at[pl.ds(row0, stripe)])
        plsc.subcore_barrier()

        def blk(i, carry):
            r = tbase + i * KJ
            pltpu.sync_copy(dst_hbm.at[pl.ds(r, KJ)], idxd)
            for j in range(KJ):
                pltpu.sync_copy(ones_v, acc.at[idxd.at[j]], add=True)
            return carry

        lax.fori_loop(0, nblk, blk, 0)
        plsc.subcore_barrier()
        pltpu.sync_copy(acc.at[pl.ds(row0, stripe)],
                        out_hbm.at[c, pl.ds(row0, stripe)])

    return pl.kernel(
        body,
        mesh=mesh,
        compiler_params=pltpu.CompilerParams(use_tc_tiling_on_sc=False),
        out_type=jax.ShapeDtypeStruct((NCORES, np_rows, 1), jnp.float32),
        scratch_types=[
            pltpu.VMEM((KJ, LANES), jnp.int32),
            pltpu.VMEM((LANES, 1), jnp.float32),
            pltpu.VMEM_SHARED((np_rows, 1), jnp.float32),
        ],
    )


@functools.lru_cache(None)
def _make_scatter(np_rows, nch, c_width, ebrows):
    rows_per_tile = ebrows // NTILES
    kj = KJ
    kh = kj // 2
    nblk = rows_per_tile // kj
    stripe = np_rows // NSUB
    mesh = plsc.VectorSubcoreMesh(core_axis_name="c", subcore_axis_name="s")

    def body(*refs):
        g_refs = refs[:nch]
        src_hbm, dst_hbm, z_hbm, out_hbm = refs[nch:nch + 4]
        idxs, idxd, rows_a, rows_b, acc, sem_a, sem_b = refs[nch + 4:]
        c = lax.axis_index("c")
        s = lax.axis_index("s")
        row0 = s * stripe
        tbase = (c * NSUB + s) * rows_per_tile

        def fire(g_hbm, j0, rows, sem):
            return [pltpu.async_copy(g_hbm.at[idxs.at[j0 + j]],
                                     rows.at[pl.ds(j * LANES, LANES)], sem)
                    for j in range(kh)]

        def scat(j0, rows):
            for j in range(kh):
                pltpu.sync_copy(rows.at[pl.ds(j * LANES, LANES)],
                                acc.at[idxd.at[j0 + j]], add=True)

        for ck in range(nch):
            g_hbm = g_refs[ck]
            pltpu.sync_copy(z_hbm, acc.at[pl.ds(row0, stripe)])
            plsc.subcore_barrier()

            def blk(i, carry):
                r = tbase + i * kj
                pltpu.sync_copy(src_hbm.at[pl.ds(r, kj)], idxs)
                pltpu.sync_copy(dst_hbm.at[pl.ds(r, kj)], idxd)
                descs_a = fire(g_hbm, 0, rows_a, sem_a)
                descs_b = fire(g_hbm, kh, rows_b, sem_b)
                for d in descs_a:
                    d.wait()
                scat(0, rows_a)
                for d in descs_b:
                    d.wait()
                scat(kh, rows_b)
                return carry

            lax.fori_loop(0, nblk, blk, 0)
            plsc.subcore_barrier()
            pltpu.sync_copy(acc.at[pl.ds(row0, stripe)],
                            out_hbm.at[c, ck, pl.ds(row0, stripe)])

    return pl.kernel(
        body,
        mesh=mesh,
        compiler_params=pltpu.CompilerParams(use_tc_tiling_on_sc=False),
        out_type=jax.ShapeDtypeStruct((NCORES, nch, np_rows, c_width),
                                      jnp.float32),
        scratch_types=[
            pltpu.VMEM((kj, LANES), jnp.int32),
            pltpu.VMEM((kj, LANES), jnp.int32),
            pltpu.VMEM((kh * LANES, c_width), jnp.float32),
            pltpu.VMEM((kh * LANES, c_width), jnp.float32),
            pltpu.VMEM_SHARED((np_rows, c_width), jnp.float32),
            pltpu.SemaphoreType.DMA,
            pltpu.SemaphoreType.DMA,
        ],
    )


def _tc_prep(deg2, x):
    n, f = x.shape
    grid = n // BLK

    def body(deg_ref, x_ref, dinv_ref, g1_ref):
        d = deg_ref[0, :, 0] + deg_ref[1, :, 0] + 1.0
        dv = lax.rsqrt(d)
        dinv_ref[:, 0] = dv
        g = x_ref[...] * dv[:, None]
        g1_ref[...] = jnp.concatenate(
            [g, jnp.zeros((BLK, 32 - f), jnp.float32)], axis=-1)

    return pl.pallas_call(
        body,
        grid=(grid,),
        in_specs=[pl.BlockSpec((NCORES, BLK, 1), lambda i: (0, i, 0)),
                  pl.BlockSpec((BLK, f), lambda i: (i, 0))],
        out_specs=[pl.BlockSpec((BLK, 1), lambda i: (i, 0)),
                   pl.BlockSpec((BLK, 32), lambda i: (i, 0))],
        out_shape=[jax.ShapeDtypeStruct((n, 1), jnp.float32),
                   jax.ShapeDtypeStruct((n, 32), jnp.float32)],
    )(deg2, x)


def _tc_layer1(s1, g1, dinv, w1p, b1r):
    n = g1.shape[0]
    grid = n // BLK

    def body(s_ref, g_ref, dv_ref, w_ref, b_ref, o0, o1, o2, o3):
        dv = dv_ref[:, 0]
        agg = (s_ref[0] + s_ref[1] + g_ref[...]) * dv[:, None]
        h = jnp.dot(agg, w_ref[...], preferred_element_type=jnp.float32,
                    precision=lax.Precision.HIGHEST) + b_ref[...]
        g2 = jnp.maximum(h, 0.0) * dv[:, None]
        o0[...] = g2[:, 0:32]
        o1[...] = g2[:, 32:64]
        o2[...] = g2[:, 64:96]
        o3[...] = g2[:, 96:128]

    return pl.pallas_call(
        body,
        grid=(grid,),
        in_specs=[pl.BlockSpec((NCORES, BLK, 32), lambda i: (0, i, 0)),
                  pl.BlockSpec((BLK, 32), lambda i: (i, 0)),
                  pl.BlockSpec((BLK, 1), lambda i: (i, 0)),
                  pl.BlockSpec((32, 128), lambda i: (0, 0)),
                  pl.BlockSpec((1, 128), lambda i: (0, 0))],
        out_specs=[pl.BlockSpec((BLK, 32), lambda i: (i, 0))] * 4,
        out_shape=[jax.ShapeDtypeStruct((n, 32), jnp.float32)] * 4,
    )(s1, g1, dinv, w1p, b1r)


def _tc_layer2(s2, g2s, dinv, w2, b2r, w3p):
    n = dinv.shape[0]
    grid = n // BLK

    def body(s_ref, g0, g1, g2, g3, dv_ref, w2_ref, b_ref, w3_ref, out_ref):
        dv = dv_ref[:, 0]
        gs = (g0, g1, g2, g3)
        parts = [s_ref[0, k] + s_ref[1, k] + gs[k][...] for k in range(4)]
        agg = jnp.concatenate(parts, axis=-1) * dv[:, None]
        h = jnp.dot(agg, w2_ref[...], preferred_element_type=jnp.float32,
                    precision=lax.Precision.HIGHEST) + b_ref[...]
        h = jnp.maximum(h, 0.0)
        p = jnp.dot(h, w3_ref[...], preferred_element_type=jnp.float32,
                    precision=lax.Precision.HIGHEST)
        out_ref[...] = p * dv[:, None]

    return pl.pallas_call(
        body,
        grid=(grid,),
        in_specs=[pl.BlockSpec((NCORES, 4, BLK, 32), lambda i: (0, 0, i, 0))]
        + [pl.BlockSpec((BLK, 32), lambda i: (i, 0))] * 4
        + [pl.BlockSpec((BLK, 1), lambda i: (i, 0)),
           pl.BlockSpec((128, 128), lambda i: (0, 0)),
           pl.BlockSpec((1, 128), lambda i: (0, 0)),
           pl.BlockSpec((128, 16), lambda i: (0, 0))],
        out_specs=pl.BlockSpec((BLK, 16), lambda i: (i, 0)),
        out_shape=jax.ShapeDtypeStruct((n, 16), jnp.float32),
    )(s2, *g2s, dinv, w2, b2r, w3p)


def _tc_out(s3, g3, dinv, b3r):
    n = dinv.shape[0]
    f_out = b3r.shape[1]
    grid = n // BLK

    def body(s_ref, g_ref, dv_ref, b_ref, out_ref):
        dv = dv_ref[:, 0]
        v = (s_ref[0] + s_ref[1] + g_ref[...]) * dv[:, None]
        out_ref[...] = v[:, :f_out] + b_ref[...]

    return pl.pallas_call(
        body,
        grid=(grid,),
        in_specs=[pl.BlockSpec((NCORES, BLK, 16), lambda i: (0, i, 0)),
                  pl.BlockSpec((BLK, 16), lambda i: (i, 0)),
                  pl.BlockSpec((BLK, 1), lambda i: (i, 0)),
                  pl.BlockSpec((1, f_out), lambda i: (0, 0))],
        out_specs=pl.BlockSpec((BLK, f_out), lambda i: (i, 0)),
        out_shape=jax.ShapeDtypeStruct((n, f_out), jnp.float32),
    )(s3, g3, dinv, b3r)


def kernel(x, edge_index, W1, b1, W2, b2, W3, b3):
    n, f_in = x.shape
    e = edge_index.shape[1]
    h = W1.shape[1]
    f_out = W3.shape[1]
    ep = _pad_to(e, NTILES * LANES)
    ebrows = ep // LANES
    pad = ep - e
    np_rows = _pad_to(n, NSUB * 8)
    stripe = np_rows // NSUB
    spread = jnp.arange(pad, dtype=jnp.int32) % NSUB
    src = jnp.concatenate([edge_index[0], spread]).reshape(ebrows, LANES)
    dst = jnp.concatenate([edge_index[1], n + spread]).reshape(ebrows, LANES)
    z1 = jnp.zeros((stripe, 1), jnp.float32)
    z32 = jnp.zeros((stripe, 32), jnp.float32)
    z16 = jnp.zeros((stripe, 16), jnp.float32)
    ones = jnp.ones((LANES, 1), jnp.float32)

    deg2 = _make_degree(np_rows, ebrows)(dst, z1, ones)
    dinv, g1 = _tc_prep(deg2, x)

    s1 = _make_scatter(np_rows, 1, 32, ebrows)(g1, src, dst, z32)
    w1p = jnp.concatenate([W1, jnp.zeros((32 - f_in, h), jnp.float32)], axis=0)
    g2s = _tc_layer1(s1[:, 0], g1, dinv, w1p, b1.reshape(1, h))

    s2 = _make_scatter(np_rows, 4, 32, ebrows)(*g2s, src, dst, z32)
    w3p = jnp.concatenate([W3, jnp.zeros((h, 16 - f_out), jnp.float32)], axis=1)
    g3 = _tc_layer2(s2, g2s, dinv, W2, b2.reshape(1, h), w3p)

    s3 = _make_scatter(np_rows, 1, 16, ebrows)(g3, src, dst, z16)
    return _tc_out(s3[:, 0], g3, dinv, b3.reshape(1, f_out))


# speedup vs baseline: 19.5563x; 1.1894x over previous
import functools

import jax
import jax.numpy as jnp
from jax import lax
from jax.experimental import pallas as pl
from jax.experimental.pallas import tpu as pltpu
from jax.experimental.pallas import tpu_sc as plsc

NCORES = 2
NSUB = 16
NTILES = NCORES * NSUB
LANES = 128
KJ = 4
KH = KJ // 2
CW = 32
M = 128 // CW
BLK = 1024
NP = 51200


def _pad_to(v, m):
    return -(-v // m) * m


@functools.lru_cache(None)
def _make_degree(ebrows):
    rows_per_tile = ebrows // NTILES
    nblk = rows_per_tile // KJ
    stripe = NP // NSUB
    mesh = plsc.VectorSubcoreMesh(core_axis_name="c", subcore_axis_name="s")

    def body(dst_hbm, z_hbm, ones_hbm, out_hbm, idxd, ones_v, acc):
        c = lax.axis_index("c")
        s = lax.axis_index("s")
        row0 = s * stripe
        tbase = (c * NSUB + s) * rows_per_tile
        pltpu.sync_copy(ones_hbm, ones_v)
        pltpu.sync_copy(z_hbm, acc.at[pl.ds(row0, stripe)])
        plsc.subcore_barrier()

        def blk(i, carry):
            r = tbase + i * KJ
            pltpu.sync_copy(dst_hbm.at[pl.ds(r, KJ)], idxd)
            for j in range(KJ):
                pltpu.sync_copy(ones_v, acc.at[idxd.at[j]], add=True)
            return carry

        lax.fori_loop(0, nblk, blk, 0)
        plsc.subcore_barrier()
        pltpu.sync_copy(acc.at[pl.ds(row0, stripe)],
                        out_hbm.at[c, pl.ds(row0, stripe)])

    return pl.kernel(
        body,
        mesh=mesh,
        compiler_params=pltpu.CompilerParams(use_tc_tiling_on_sc=False),
        out_type=jax.ShapeDtypeStruct((NCORES, NP, 1), jnp.float32),
        scratch_types=[
            pltpu.VMEM((KJ, LANES), jnp.int32),
            pltpu.VMEM((LANES, 1), jnp.float32),
            pltpu.VMEM_SHARED((NP, 1), jnp.float32),
        ],
    )


@functools.lru_cache(None)
def _make_scatter(nck, ebrows):
    rows_per_tile = ebrows // NTILES
    nblk = rows_per_tile // KJ
    stripe = NP // NSUB
    mesh = plsc.VectorSubcoreMesh(core_axis_name="c", subcore_axis_name="s")

    def body(g_hbm, src_hbm, dst_hbm, z_hbm, out_hbm,
             idxs, idxd, idxg, rows_a, rows_b, acc, sem_a, sem_b):
        c = lax.axis_index("c")
        s = lax.axis_index("s")
        row0 = s * stripe
        tbase = (c * NSUB + s) * rows_per_tile

        def fire(j0, rows, sem):
            return [pltpu.async_copy(g_hbm.at[idxg.at[j0 + j]],
                                     rows.at[pl.ds(j * LANES, LANES)], sem)
                    for j in range(KH)]

        def scat(j0, rows):
            for j in range(KH):
                pltpu.sync_copy(rows.at[pl.ds(j * LANES, LANES)],
                                acc.at[idxd.at[j0 + j]], add=True)

        for ck in range(nck):
            pltpu.sync_copy(z_hbm, acc.at[pl.ds(row0, stripe)])
            plsc.subcore_barrier()

            def blk(i, carry, ck=ck):
                r = tbase + i * KJ
                pltpu.sync_copy(src_hbm.at[pl.ds(r, KJ)], idxs)
                pltpu.sync_copy(dst_hbm.at[pl.ds(r, KJ)], idxd)
                for j in range(KJ):
                    for v in range(LANES // 16):
                        sl = pl.ds(v * 16, 16)
                        idxg[j, sl] = idxs[j, sl] * M + ck
                descs_a = fire(0, rows_a, sem_a)
                descs_b = fire(KH, rows_b, sem_b)
                for d in descs_a:
                    d.wait()
                scat(0, rows_a)
                for d in descs_b:
                    d.wait()
                scat(KH, rows_b)
                return carry

            lax.fori_loop(0, nblk, blk, 0)
            plsc.subcore_barrier()
            pltpu.sync_copy(acc.at[pl.ds(row0, stripe)],
                            out_hbm.at[c, pl.ds(row0, stripe),
                                       pl.ds(ck * CW, CW)])

    return pl.kernel(
        body,
        mesh=mesh,
        compiler_params=pltpu.CompilerParams(use_tc_tiling_on_sc=False),
        out_type=jax.ShapeDtypeStruct((NCORES, NP, 128), jnp.float32),
        scratch_types=[
            pltpu.VMEM((KJ, LANES), jnp.int32),
            pltpu.VMEM((KJ, LANES), jnp.int32),
            pltpu.VMEM((KJ, LANES), jnp.int32),
            pltpu.VMEM((KH * LANES, CW), jnp.float32),
            pltpu.VMEM((KH * LANES, CW), jnp.float32),
            pltpu.VMEM_SHARED((NP, CW), jnp.float32),
            pltpu.SemaphoreType.DMA,
            pltpu.SemaphoreType.DMA,
        ],
    )


def _tc_prep(deg2, xp):
    f = xp.shape[1]
    grid = NP // BLK

    def body(deg_ref, x_ref, dinv_ref, g1_ref):
        d = deg_ref[0, :, 0] + deg_ref[1, :, 0] + 1.0
        dv = lax.rsqrt(d)
        dinv_ref[:, 0] = dv
        g = x_ref[...] * dv[:, None]
        g1_ref[...] = jnp.concatenate(
            [g, jnp.zeros((BLK, 128 - f), jnp.float32)], axis=-1)

    return pl.pallas_call(
        body,
        grid=(grid,),
        in_specs=[pl.BlockSpec((NCORES, BLK, 1), lambda i: (0, i, 0)),
                  pl.BlockSpec((BLK, f), lambda i: (i, 0))],
        out_specs=[pl.BlockSpec((BLK, 1), lambda i: (i, 0)),
                   pl.BlockSpec((BLK, 128), lambda i: (i, 0))],
        out_shape=[jax.ShapeDtypeStruct((NP, 1), jnp.float32),
                   jax.ShapeDtypeStruct((NP, 128), jnp.float32)],
    )(deg2, xp)


def _tc_layer1(s1, g1, dinv, w1p, b1r):
    def body(s_ref, g_ref, dv_ref, w_ref, b_ref, out_ref):
        dv = dv_ref[:, 0]
        agg = (s_ref[0, :, :CW] + s_ref[1, :, :CW] + g_ref[:, :CW]) \
            * dv[:, None]
        h = jnp.dot(agg, w_ref[...], preferred_element_type=jnp.float32,
                    precision=lax.Precision.HIGHEST) + b_ref[...]
        out_ref[...] = jnp.maximum(h, 0.0) * dv[:, None]

    return pl.pallas_call(
        body,
        grid=(NP // BLK,),
        in_specs=[pl.BlockSpec((NCORES, BLK, 128), lambda i: (0, i, 0)),
                  pl.BlockSpec((BLK, 128), lambda i: (i, 0)),
                  pl.BlockSpec((BLK, 1), lambda i: (i, 0)),
                  pl.BlockSpec((CW, 128), lambda i: (0, 0)),
                  pl.BlockSpec((1, 128), lambda i: (0, 0))],
        out_specs=pl.BlockSpec((BLK, 128), lambda i: (i, 0)),
        out_shape=jax.ShapeDtypeStruct((NP, 128), jnp.float32),
    )(s1, g1, dinv, w1p, b1r)


def _tc_layer2(s2, g2, dinv, w2, b2r, w3p):
    def body(s_ref, g_ref, dv_ref, w2_ref, b_ref, w3_ref, out_ref):
        dv = dv_ref[:, 0]
        agg = (s_ref[0] + s_ref[1] + g_ref[...]) * dv[:, None]
        h = jnp.dot(agg, w2_ref[...], preferred_element_type=jnp.float32,
                    precision=lax.Precision.HIGHEST) + b_ref[...]
        h = jnp.maximum(h, 0.0)
        p = jnp.dot(h, w3_ref[...], preferred_element_type=jnp.float32,
                    precision=lax.Precision.HIGHEST)
        g3 = p * dv[:, None]
        out_ref[...] = jnp.concatenate(
            [g3, jnp.zeros((BLK, 112), jnp.float32)], axis=-1)

    return pl.pallas_call(
        body,
        grid=(NP // BLK,),
        in_specs=[pl.BlockSpec((NCORES, BLK, 128), lambda i: (0, i, 0)),
                  pl.BlockSpec((BLK, 128), lambda i: (i, 0)),
                  pl.BlockSpec((BLK, 1), lambda i: (i, 0)),
                  pl.BlockSpec((128, 128), lambda i: (0, 0)),
                  pl.BlockSpec((1, 128), lambda i: (0, 0)),
                  pl.BlockSpec((128, 16), lambda i: (0, 0))],
        out_specs=pl.BlockSpec((BLK, 128), lambda i: (i, 0)),
        out_shape=jax.ShapeDtypeStruct((NP, 128), jnp.float32),
    )(s2, g2, dinv, w2, b2r, w3p)


def _tc_out(s3, g3, dinv, b3r):
    f_out = b3r.shape[1]

    def body(s_ref, g_ref, dv_ref, b_ref, out_ref):
        dv = dv_ref[:, 0]
        v = (s_ref[0, :, :16] + s_ref[1, :, :16] + g_ref[:, :16]) \
            * dv[:, None]
        out_ref[...] = v[:, :f_out] + b_ref[...]

    return pl.pallas_call(
        body,
        grid=(NP // BLK,),
        in_specs=[pl.BlockSpec((NCORES, BLK, 128), lambda i: (0, i, 0)),
                  pl.BlockSpec((BLK, 128), lambda i: (i, 0)),
                  pl.BlockSpec((BLK, 1), lambda i: (i, 0)),
                  pl.BlockSpec((1, f_out), lambda i: (0, 0))],
        out_specs=pl.BlockSpec((BLK, f_out), lambda i: (i, 0)),
        out_shape=jax.ShapeDtypeStruct((NP, f_out), jnp.float32),
    )(s3, g3, dinv, b3r)


def kernel(x, edge_index, W1, b1, W2, b2, W3, b3):
    n, f_in = x.shape
    e = edge_index.shape[1]
    h = W1.shape[1]
    f_out = W3.shape[1]
    ep = _pad_to(e, NTILES * LANES)
    ebrows = ep // LANES
    pad = ep - e
    stripe = NP // NSUB
    spread = jnp.arange(pad, dtype=jnp.int32) % NSUB
    src = jnp.concatenate([edge_index[0], spread]).reshape(ebrows, LANES)
    dst = jnp.concatenate([edge_index[1], n + spread]).reshape(ebrows, LANES)
    xp = jnp.pad(x, ((0, NP - n), (0, 0)))
    z1 = jnp.zeros((stripe, 1), jnp.float32)
    z32 = jnp.zeros((stripe, CW), jnp.float32)
    ones = jnp.ones((LANES, 1), jnp.float32)

    deg2 = _make_degree(ebrows)(dst, z1, ones)
    dinv, g1 = _tc_prep(deg2, xp)

    s1 = _make_scatter(1, ebrows)(g1.reshape(M * NP, CW), src, dst, z32)
    w1p = jnp.concatenate([W1, jnp.zeros((CW - f_in, h), jnp.float32)], axis=0)
    g2 = _tc_layer1(s1, g1, dinv, w1p, b1.reshape(1, h))

    s2 = _make_scatter(4, ebrows)(g2.reshape(M * NP, CW), src, dst, z32)
    w3p = jnp.concatenate([W3, jnp.zeros((h, 16 - f_out), jnp.float32)], axis=1)
    g3 = _tc_layer2(s2, g2, dinv, W2, b2.reshape(1, h), w3p)

    s3 = _make_scatter(1, ebrows)(g3.reshape(M * NP, CW), src, dst, z32)
    out = _tc_out(s3, g3, dinv, b3.reshape(1, f_out))
    return out[:n]


# speedup vs baseline: 22.5001x; 1.1505x over previous
import functools

import jax
import jax.numpy as jnp
from jax import lax
from jax.experimental import pallas as pl
from jax.experimental.pallas import tpu as pltpu
from jax.experimental.pallas import tpu_sc as plsc

NCORES = 2
NSUB = 16
NTILES = NCORES * NSUB
LANES = 128
KJ = 4
KH = KJ // 2
CW = 32
M = 128 // CW
BLK = 1024
NP = 51200


def _pad_to(v, m):
    return -(-v // m) * m


@functools.lru_cache(None)
def _make_degree(ebrows):
    rows_per_tile = ebrows // NTILES
    nblk = rows_per_tile // KJ
    stripe = NP // NSUB
    mesh = plsc.VectorSubcoreMesh(core_axis_name="c", subcore_axis_name="s")

    def body(dst_hbm, z_hbm, ones_hbm, out_hbm, idxd, ones_v, acc):
        c = lax.axis_index("c")
        s = lax.axis_index("s")
        row0 = s * stripe
        tbase = (c * NSUB + s) * rows_per_tile
        pltpu.sync_copy(ones_hbm, ones_v)
        pltpu.sync_copy(z_hbm, acc.at[pl.ds(row0, stripe)])
        plsc.subcore_barrier()

        def blk(i, carry):
            r = tbase + i * KJ
            pltpu.sync_copy(dst_hbm.at[pl.ds(r, KJ)], idxd)
            for j in range(KJ):
                pltpu.sync_copy(ones_v, acc.at[idxd.at[j]], add=True)
            return carry

        lax.fori_loop(0, nblk, blk, 0)
        plsc.subcore_barrier()
        pltpu.sync_copy(acc.at[pl.ds(row0, stripe)],
                        out_hbm.at[c, pl.ds(row0, stripe)])

    return pl.kernel(
        body,
        mesh=mesh,
        compiler_params=pltpu.CompilerParams(use_tc_tiling_on_sc=False),
        out_type=jax.ShapeDtypeStruct((NCORES, NP, 1), jnp.float32),
        scratch_types=[
            pltpu.VMEM((KJ, LANES), jnp.int32),
            pltpu.VMEM((LANES, 1), jnp.float32),
            pltpu.VMEM_SHARED((NP, 1), jnp.float32),
        ],
    )


@functools.lru_cache(None)
def _make_scatter(nck, ebrows):
    rows_per_tile = ebrows // NTILES
    nblk = rows_per_tile // KJ
    sb = 4
    nsb = nblk // sb
    tail = nblk - nsb * sb
    stripe = NP // NSUB
    mesh = plsc.VectorSubcoreMesh(core_axis_name="c", subcore_axis_name="s")

    def body(g_hbm, src_hbm, dst_hbm, z_hbm, out_hbm,
             idxs, idxd, idxg, idxd4, rows_a, rows_b, acc, sem_a, sem_b):
        c = lax.axis_index("c")
        s = lax.axis_index("s")
        row0 = s * stripe
        tbase = (c * NSUB + s) * rows_per_tile

        def fire(j0, rows, sem):
            return [pltpu.async_copy(g_hbm.at[idxg.at[j0 + j]],
                                     rows.at[pl.ds(j * LANES, LANES)], sem)
                    for j in range(KH)]

        def scat(j0, rows):
            for j in range(KH):
                pltpu.sync_copy(rows.at[pl.ds(j * LANES, LANES)],
                                acc.at[idxd4.at[j0 + j]], add=True)

        def run_blocks(nb, ck):
            for ib in range(nb):
                q0 = ib * KJ
                for j in range(KJ):
                    for v in range(LANES // 16):
                        sl = pl.ds(v * 16, 16)
                        idxg[j, sl] = idxs[q0 + j, sl] * M + ck
                        idxd4[j, sl] = idxd[q0 + j, sl]
                descs_a = fire(0, rows_a, sem_a)
                descs_b = fire(KH, rows_b, sem_b)
                for d in descs_a:
                    d.wait()
                scat(0, rows_a)
                for d in descs_b:
                    d.wait()
                scat(KH, rows_b)

        for ck in range(nck):
            pltpu.sync_copy(z_hbm, acc.at[pl.ds(row0, stripe)])
            plsc.subcore_barrier()

            def blk(i, carry, ck=ck):
                r = tbase + i * sb * KJ
                pltpu.sync_copy(src_hbm.at[pl.ds(r, sb * KJ)], idxs)
                pltpu.sync_copy(dst_hbm.at[pl.ds(r, sb * KJ)], idxd)
                run_blocks(sb, ck)
                return carry

            lax.fori_loop(0, nsb, blk, 0)
            if tail:
                r = tbase + nsb * sb * KJ
                pltpu.sync_copy(src_hbm.at[pl.ds(r, tail * KJ)],
                                idxs.at[pl.ds(0, tail * KJ)])
                pltpu.sync_copy(dst_hbm.at[pl.ds(r, tail * KJ)],
                                idxd.at[pl.ds(0, tail * KJ)])
                run_blocks(tail, ck)
            plsc.subcore_barrier()
            pltpu.sync_copy(acc.at[pl.ds(row0, stripe)],
                            out_hbm.at[c, pl.ds(row0, stripe),
                                       pl.ds(ck * CW, CW)])

    return pl.kernel(
        body,
        mesh=mesh,
        compiler_params=pltpu.CompilerParams(use_tc_tiling_on_sc=False),
        out_type=jax.ShapeDtypeStruct((NCORES, NP, 128), jnp.float32),
        scratch_types=[
            pltpu.VMEM((4 * KJ, LANES), jnp.int32),
            pltpu.VMEM((4 * KJ, LANES), jnp.int32),
            pltpu.VMEM((KJ, LANES), jnp.int32),
            pltpu.VMEM((KJ, LANES), jnp.int32),
            pltpu.VMEM((KH * LANES, CW), jnp.float32),
            pltpu.VMEM((KH * LANES, CW), jnp.float32),
            pltpu.VMEM_SHARED((NP, CW), jnp.float32),
            pltpu.SemaphoreType.DMA,
            pltpu.SemaphoreType.DMA,
        ],
    )


def _tc_prep(deg2, xp):
    f = xp.shape[1]
    grid = NP // BLK

    def body(deg_ref, x_ref, dinv_ref, g1_ref):
        d = deg_ref[0, :, 0] + deg_ref[1, :, 0] + 1.0
        dv = lax.rsqrt(d)
        dinv_ref[:, 0] = dv
        g = x_ref[...] * dv[:, None]
        g1_ref[...] = jnp.concatenate(
            [g, jnp.zeros((BLK, 128 - f), jnp.float32)], axis=-1)

    return pl.pallas_call(
        body,
        grid=(grid,),
        in_specs=[pl.BlockSpec((NCORES, BLK, 1), lambda i: (0, i, 0)),
                  pl.BlockSpec((BLK, f), lambda i: (i, 0))],
        out_specs=[pl.BlockSpec((BLK, 1), lambda i: (i, 0)),
                   pl.BlockSpec((BLK, 128), lambda i: (i, 0))],
        out_shape=[jax.ShapeDtypeStruct((NP, 1), jnp.float32),
                   jax.ShapeDtypeStruct((NP, 128), jnp.float32)],
    )(deg2, xp)


def _tc_layer1(s1, g1, dinv, w1p, b1r):
    def body(s_ref, g_ref, dv_ref, w_ref, b_ref, out_ref):
        dv = dv_ref[:, 0]
        agg = (s_ref[0, :, :CW] + s_ref[1, :, :CW] + g_ref[:, :CW]) \
            * dv[:, None]
        h = jnp.dot(agg, w_ref[...], preferred_element_type=jnp.float32,
                    precision=lax.Precision.HIGHEST) + b_ref[...]
        out_ref[...] = jnp.maximum(h, 0.0) * dv[:, None]

    return pl.pallas_call(
        body,
        grid=(NP // BLK,),
        in_specs=[pl.BlockSpec((NCORES, BLK, 128), lambda i: (0, i, 0)),
                  pl.BlockSpec((BLK, 128), lambda i: (i, 0)),
                  pl.BlockSpec((BLK, 1), lambda i: (i, 0)),
                  pl.BlockSpec((CW, 128), lambda i: (0, 0)),
                  pl.BlockSpec((1, 128), lambda i: (0, 0))],
        out_specs=pl.BlockSpec((BLK, 128), lambda i: (i, 0)),
        out_shape=jax.ShapeDtypeStruct((NP, 128), jnp.float32),
    )(s1, g1, dinv, w1p, b1r)


def _tc_layer2(s2, g2, dinv, w2, b2r, w3p):
    def body(s_ref, g_ref, dv_ref, w2_ref, b_ref, w3_ref, out_ref):
        dv = dv_ref[:, 0]
        agg = (s_ref[0] + s_ref[1] + g_ref[...]) * dv[:, None]
        h = jnp.dot(agg, w2_ref[...], preferred_element_type=jnp.float32,
                    precision=lax.Precision.HIGHEST) + b_ref[...]
        h = jnp.maximum(h, 0.0)
        p = jnp.dot(h, w3_ref[...], preferred_element_type=jnp.float32,
                    precision=lax.Precision.HIGHEST)
        g3 = p * dv[:, None]
        out_ref[...] = jnp.concatenate(
            [g3, jnp.zeros((BLK, 112), jnp.float32)], axis=-1)

    return pl.pallas_call(
        body,
        grid=(NP // BLK,),
        in_specs=[pl.BlockSpec((NCORES, BLK, 128), lambda i: (0, i, 0)),
                  pl.BlockSpec((BLK, 128), lambda i: (i, 0)),
                  pl.BlockSpec((BLK, 1), lambda i: (i, 0)),
                  pl.BlockSpec((128, 128), lambda i: (0, 0)),
                  pl.BlockSpec((1, 128), lambda i: (0, 0)),
                  pl.BlockSpec((128, 16), lambda i: (0, 0))],
        out_specs=pl.BlockSpec((BLK, 128), lambda i: (i, 0)),
        out_shape=jax.ShapeDtypeStruct((NP, 128), jnp.float32),
    )(s2, g2, dinv, w2, b2r, w3p)


def _tc_out(s3, g3, dinv, b3r):
    f_out = b3r.shape[1]

    def body(s_ref, g_ref, dv_ref, b_ref, out_ref):
        dv = dv_ref[:, 0]
        v = (s_ref[0, :, :16] + s_ref[1, :, :16] + g_ref[:, :16]) \
            * dv[:, None]
        out_ref[...] = v[:, :f_out] + b_ref[...]

    return pl.pallas_call(
        body,
        grid=(NP // BLK,),
        in_specs=[pl.BlockSpec((NCORES, BLK, 128), lambda i: (0, i, 0)),
                  pl.BlockSpec((BLK, 128), lambda i: (i, 0)),
                  pl.BlockSpec((BLK, 1), lambda i: (i, 0)),
                  pl.BlockSpec((1, f_out), lambda i: (0, 0))],
        out_specs=pl.BlockSpec((BLK, f_out), lambda i: (i, 0)),
        out_shape=jax.ShapeDtypeStruct((NP, f_out), jnp.float32),
    )(s3, g3, dinv, b3r)


def kernel(x, edge_index, W1, b1, W2, b2, W3, b3):
    n, f_in = x.shape
    e = edge_index.shape[1]
    h = W1.shape[1]
    f_out = W3.shape[1]
    ep = _pad_to(e, NTILES * LANES)
    ebrows = ep // LANES
    pad = ep - e
    stripe = NP // NSUB
    spread = jnp.arange(pad, dtype=jnp.int32) % NSUB
    src = jnp.concatenate([edge_index[0], spread]).reshape(ebrows, LANES)
    dst = jnp.concatenate([edge_index[1], n + spread]).reshape(ebrows, LANES)
    xp = jnp.pad(x, ((0, NP - n), (0, 0)))
    z1 = jnp.zeros((stripe, 1), jnp.float32)
    z32 = jnp.zeros((stripe, CW), jnp.float32)
    ones = jnp.ones((LANES, 1), jnp.float32)

    deg2 = _make_degree(ebrows)(dst, z1, ones)
    dinv, g1 = _tc_prep(deg2, xp)

    s1 = _make_scatter(1, ebrows)(g1.reshape(M * NP, CW), src, dst, z32)
    w1p = jnp.concatenate([W1, jnp.zeros((CW - f_in, h), jnp.float32)], axis=0)
    g2 = _tc_layer1(s1, g1, dinv, w1p, b1.reshape(1, h))

    s2 = _make_scatter(4, ebrows)(g2.reshape(M * NP, CW), src, dst, z32)
    w3p = jnp.concatenate([W3, jnp.zeros((h, 16 - f_out), jnp.float32)], axis=1)
    g3 = _tc_layer2(s2, g2, dinv, W2, b2.reshape(1, h), w3p)

    s3 = _make_scatter(1, ebrows)(g3.reshape(M * NP, CW), src, dst, z32)
    out = _tc_out(s3, g3, dinv, b3.reshape(1, f_out))
    return out[:n]
